# baseline (speedup 1.0000x reference)
import functools

import jax
import jax.numpy as jnp
from jax import lax
from jax.experimental import pallas as pl
from jax.experimental.pallas import tpu as pltpu
from jax.experimental.pallas import tpu_sc as plsc

N, E, D, H, Z, G = 10000, 320000, 128, 64, 32, 64

NC, NS = 2, 16
NW = NC * NS
TW = H + 8
NP = N
CH = 40
EW = E // NW
NCHUNK = EW // CH
NBUF = 10
NOUTER = NCHUNK // NBUF
ZR = NP // NS


def _table_body(x_ref, w1_ref, o_ref):
    xw = jnp.dot(x_ref[...], w1_ref[...], preferred_element_type=jnp.float32)
    col = lax.broadcasted_iota(jnp.int32, (N, TW - H), 1)
    pad = jnp.where(col == 0, 1.0, 0.0).astype(jnp.float32)
    o_ref[...] = jnp.concatenate([xw, pad], axis=1)


def _sc_body(table_hbm, srcm_hbm, dstm_hbm, zeros_hbm, out_hbm,
             srcv, dstv, rows, acc, gsem, ssem, isem):
    cid = lax.axis_index("c")
    sid = lax.axis_index("s")
    wid = sid * NC + cid

    zd = pltpu.async_copy(zeros_hbm, acc.at[pl.ds(sid * ZR, ZR)], isem.at[0])
    sd = pltpu.async_copy(srcm_hbm.at[pl.ds(wid * NCHUNK, NCHUNK)], srcv,
                          isem.at[1])
    dd = pltpu.async_copy(dstm_hbm.at[pl.ds(wid * NCHUNK, NCHUNK)], dstv,
                          isem.at[2])

    def start_gather(j, b):
        pltpu.async_copy(table_hbm.at[srcv.at[j]], rows.at[b], gsem.at[b])

    def wait_gather(j, b):
        pltpu.make_async_copy(table_hbm.at[srcv.at[j]], rows.at[b],
                              gsem.at[b]).wait()

    def start_scatter(j, b):
        pltpu.async_copy(rows.at[b], acc.at[dstv.at[j]], ssem.at[b], add=True)

    def wait_scatter(j, b):
        pltpu.make_async_copy(rows.at[b], acc.at[dstv.at[j]],
                              ssem.at[b]).wait()

    sd.wait()
    for b in range(NBUF):
        start_gather(b, b)
    dd.wait()
    zd.wait()
    plsc.subcore_barrier()

    def outer(g, carry):
        for b in range(NBUF):
            j = g * NBUF + b
            wait_gather(j, b)
            start_scatter(j, b)
        for b in range(NBUF):
            j = g * NBUF + b
            wait_scatter(j, b)

            @pl.when(g < NOUTER - 1)
            def _():
                start_gather(j + NBUF, b)
        return carry

    lax.fori_loop(0, NOUTER, outer, 0)
    plsc.subcore_barrier()
    pltpu.sync_copy(acc.at[pl.ds(sid * ZR, ZR)],
                    out_hbm.at[pl.ds(cid * NP + sid * ZR, ZR)])


@functools.lru_cache(maxsize=1)
def _make_sc_call():
    return pl.kernel(
        _sc_body,
        out_type=jax.ShapeDtypeStruct((NC * NP, TW), jnp.float32),
        mesh=plsc.VectorSubcoreMesh(core_axis_name="c", subcore_axis_name="s",
                                    num_cores=NC, num_subcores=NS),
        scratch_types=[
            pltpu.VMEM((NCHUNK, CH), jnp.int32),
            pltpu.VMEM((NCHUNK, CH), jnp.int32),
            pltpu.VMEM((NBUF, CH, TW), jnp.float32),
            pltpu.VMEM_SHARED((NP, TW), jnp.float32),
            pltpu.SemaphoreType.DMA((NBUF,)),
            pltpu.SemaphoreType.DMA((NBUF,)),
            pltpu.SemaphoreType.DMA((3,)),
        ],
        compiler_params=pltpu.CompilerParams(use_tc_tiling_on_sc=False),
    )


def _final_body(table_ref, acc_ref, x_ref, b1_ref, wmu_ref, wdecp_ref,
                bdecp_ref, batch_ref, o_ref):
    accs = acc_ref[0:N, :] + acc_ref[NP:NP + N, :]
    seg = accs[:, 0:H]
    deg = accs[:, H:H + 1]
    xw = table_ref[0:N, 0:H]
    h = jnp.maximum(xw + seg / jnp.maximum(deg, 1.0) + b1_ref[...], 0.0)
    wc = jnp.dot(wmu_ref[...], wdecp_ref[...],
                 preferred_element_type=jnp.float32)
    colmask = (lax.broadcasted_iota(jnp.int32, (1, D), 1) > 0
               ).astype(jnp.float32)
    e = (jnp.dot(h, wc, preferred_element_type=jnp.float32)
         + bdecp_ref[...] - x_ref[...] * colmask)
    ne = jnp.sum(e * e, axis=1, keepdims=True) * (1.0 / (D - 1))
    onehot = (batch_ref[...] == lax.broadcasted_iota(jnp.int32, (N, G), 1)
              ).astype(jnp.float32)
    ssum = jnp.sum(ne * onehot, axis=0, keepdims=True)
    cnt = jnp.sum(onehot, axis=0, keepdims=True)
    ge = ssum / jnp.maximum(cnt, 1.0)
    o_ref[...] = jnp.concatenate([-ge, ge], axis=0)


def kernel(x, edge_index, batch, W1, b1, Wmu, Wdec, bdec):
    table = pl.pallas_call(
        _table_body,
        out_shape=jax.ShapeDtypeStruct((NP, TW), jnp.float32),
    )(x, W1)

    srcm = edge_index[0].reshape(NW * NCHUNK, CH)
    dstm = edge_index[1].reshape(NW * NCHUNK, CH)
    zeros_blk = jnp.zeros((ZR, TW), jnp.float32)
    acc = _make_sc_call()(table, srcm, dstm, zeros_blk)

    wdecp = jnp.pad(Wdec, ((0, 0), (1, 0)))
    bdecp = jnp.pad(bdec, (1, 0)).reshape(1, D)
    out2 = pl.pallas_call(
        _final_body,
        out_shape=jax.ShapeDtypeStruct((2, G), jnp.float32),
    )(table, acc, x, b1.reshape(1, H), Wmu, wdecp, bdecp,
      batch.reshape(N, 1))
    return out2.T


# speedup vs baseline: 1.0014x; 1.0014x over previous
import functools

import jax
import jax.numpy as jnp
from jax import lax
from jax.experimental import pallas as pl
from jax.experimental.pallas import tpu as pltpu
from jax.experimental.pallas import tpu_sc as plsc

N, E, D, H, Z, G = 10000, 320000, 128, 64, 32, 64

NC, NS = 2, 16
NW = NC * NS
TW = H + 8
NP = N
CH = 80
EW = E // NW
NCHUNK = EW // CH
NBUF = 5
NOUTER = NCHUNK // NBUF
ZR = NP // NS


def _table_body(x_ref, w1_ref, o_ref):
    xw = jnp.dot(x_ref[...], w1_ref[...], preferred_element_type=jnp.float32)
    col = lax.broadcasted_iota(jnp.int32, (N, TW - H), 1)
    pad = jnp.where(col == 0, 1.0, 0.0).astype(jnp.float32)
    o_ref[...] = jnp.concatenate([xw, pad], axis=1)


def _sc_body(table_hbm, srcm_hbm, dstm_hbm, zeros_hbm, out_hbm,
             srcv, dstv, rows, acc, gsem, ssem):
    cid = lax.axis_index("c")
    sid = lax.axis_index("s")
    wid = sid * NC + cid

    pltpu.sync_copy(zeros_hbm, acc.at[pl.ds(sid * ZR, ZR)])
    pltpu.sync_copy(srcm_hbm.at[pl.ds(wid * NCHUNK, NCHUNK)], srcv)
    pltpu.sync_copy(dstm_hbm.at[pl.ds(wid * NCHUNK, NCHUNK)], dstv)
    plsc.subcore_barrier()

    def start_gather(j, b):
        pltpu.async_copy(table_hbm.at[srcv.at[j]], rows.at[b], gsem.at[b])

    def wait_gather(j, b):
        pltpu.make_async_copy(table_hbm.at[srcv.at[j]], rows.at[b],
                              gsem.at[b]).wait()

    def start_scatter(j, b):
        pltpu.async_copy(rows.at[b], acc.at[dstv.at[j]], ssem.at[b], add=True)

    def wait_scatter(j, b):
        pltpu.make_async_copy(rows.at[b], acc.at[dstv.at[j]],
                              ssem.at[b]).wait()

    for b in range(NBUF):
        start_gather(b, b)

    def outer(g, carry):
        for b in range(NBUF):
            j = g * NBUF + b
            wait_gather(j, b)
            start_scatter(j, b)
        for b in range(NBUF):
            j = g * NBUF + b
            wait_scatter(j, b)

            @pl.when(g < NOUTER - 1)
            def _():
                start_gather(j + NBUF, b)
        return carry

    lax.fori_loop(0, NOUTER, outer, 0)
    plsc.subcore_barrier()
    pltpu.sync_copy(acc.at[pl.ds(sid * ZR, ZR)],
                    out_hbm.at[pl.ds(cid * NP + sid * ZR, ZR)])


@functools.lru_cache(maxsize=1)
def _make_sc_call():
    return pl.kernel(
        _sc_body,
        out_type=jax.ShapeDtypeStruct((NC * NP, TW), jnp.float32),
        mesh=plsc.VectorSubcoreMesh(core_axis_name="c", subcore_axis_name="s",
                                    num_cores=NC, num_subcores=NS),
        scratch_types=[
            pltpu.VMEM((NCHUNK, CH), jnp.int32),
            pltpu.VMEM((NCHUNK, CH), jnp.int32),
            pltpu.VMEM((NBUF, CH, TW), jnp.float32),
            pltpu.VMEM_SHARED((NP, TW), jnp.float32),
            pltpu.SemaphoreType.DMA((NBUF,)),
            pltpu.SemaphoreType.DMA((NBUF,)),
        ],
        compiler_params=pltpu.CompilerParams(use_tc_tiling_on_sc=False),
    )


def _final_body(table_ref, acc_ref, x_ref, b1_ref, wmu_ref, wdecp_ref,
                bdecp_ref, batch_ref, o_ref):
    accs = acc_ref[0:N, :] + acc_ref[NP:NP + N, :]
    seg = accs[:, 0:H]
    deg = accs[:, H:H + 1]
    xw = table_ref[0:N, 0:H]
    h = jnp.maximum(xw + seg / jnp.maximum(deg, 1.0) + b1_ref[...], 0.0)
    wc = jnp.dot(wmu_ref[...], wdecp_ref[...],
                 preferred_element_type=jnp.float32)
    colmask = (lax.broadcasted_iota(jnp.int32, (1, D), 1) > 0
               ).astype(jnp.float32)
    e = (jnp.dot(h, wc, preferred_element_type=jnp.float32)
         + bdecp_ref[...] - x_ref[...] * colmask)
    ne = jnp.sum(e * e, axis=1, keepdims=True) * (1.0 / (D - 1))
    onehot = (batch_ref[...] == lax.broadcasted_iota(jnp.int32, (N, G), 1)
              ).astype(jnp.float32)
    ssum = jnp.sum(ne * onehot, axis=0, keepdims=True)
    cnt = jnp.sum(onehot, axis=0, keepdims=True)
    ge = ssum / jnp.maximum(cnt, 1.0)
    o_ref[...] = jnp.concatenate([-ge, ge], axis=0)


def kernel(x, edge_index, batch, W1, b1, Wmu, Wdec, bdec):
    table = pl.pallas_call(
        _table_body,
        out_shape=jax.ShapeDtypeStruct((NP, TW), jnp.float32),
    )(x, W1)

    srcm = edge_index[0].reshape(NW * NCHUNK, CH)
    dstm = edge_index[1].reshape(NW * NCHUNK, CH)
    zeros_blk = jnp.zeros((ZR, TW), jnp.float32)
    acc = _make_sc_call()(table, srcm, dstm, zeros_blk)

    wdecp = jnp.pad(Wdec, ((0, 0), (1, 0)))
    bdecp = jnp.pad(bdec, (1, 0)).reshape(1, D)
    out2 = pl.pallas_call(
        _final_body,
        out_shape=jax.ShapeDtypeStruct((2, G), jnp.float32),
    )(table, acc, x, b1.reshape(1, H), Wmu, wdecp, bdecp,
      batch.reshape(N, 1))
    return out2.T
